# Initial kernel scaffold; baseline (speedup 1.0000x reference)
import functools

import jax
import jax.numpy as jnp
from jax import lax
from jax.experimental import pallas as pl
from jax.experimental.pallas import tpu as pltpu
from jax.experimental.pallas import tpu_sc as plsc

NC = 2
NS = 16
CHUNK = 128


def _sc_gather_body(nchunks, h0, rel_t, vi, rl, hv_out, rv_out,
                    vi_v, rl_v, hv_v, rv_v, sem1, sem2):
    c = lax.axis_index("c")
    s = lax.axis_index("s")
    wid = c * NS + s
    nw = NC * NS

    def do_chunk(cid):
        off = cid * CHUNK
        pltpu.sync_copy(vi.at[pl.ds(off, CHUNK)], vi_v)
        pltpu.sync_copy(rl.at[pl.ds(off, CHUNK)], rl_v)
        cp1 = pltpu.async_copy(h0.at[vi_v], hv_v, sem1)
        cp2 = pltpu.async_copy(rel_t.at[rl_v], rv_v, sem2)
        cp1.wait()
        cp2.wait()
        pltpu.sync_copy(hv_v, hv_out.at[pl.ds(off, CHUNK)])
        pltpu.sync_copy(rv_v, rv_out.at[pl.ds(off, CHUNK)])

    rounds = nchunks // nw
    rem = nchunks - rounds * nw

    def body(k, carry):
        do_chunk(wid + nw * k)
        return carry

    lax.fori_loop(0, rounds, body, 0)

    @pl.when(wid < rem)
    def _():
        do_chunk(rounds * nw + wid)


def _sc_gather(h0, rel_emb, vi, rl):
    n, d = h0.shape
    e = vi.shape[0]
    assert e % CHUNK == 0
    nchunks = e // CHUNK
    mesh = plsc.VectorSubcoreMesh(core_axis_name="c", subcore_axis_name="s")
    return pl.kernel(
        functools.partial(_sc_gather_body, nchunks),
        out_type=[jax.ShapeDtypeStruct((e, d), jnp.float32),
                  jax.ShapeDtypeStruct((e, d), jnp.float32)],
        mesh=mesh,
        scratch_types=[
            pltpu.VMEM((CHUNK,), jnp.int32),
            pltpu.VMEM((CHUNK,), jnp.int32),
            pltpu.VMEM((CHUNK, d), jnp.float32),
            pltpu.VMEM((CHUNK, d), jnp.float32),
            pltpu.SemaphoreType.DMA,
            pltpu.SemaphoreType.DMA,
        ],
    )(h0, rel_emb, vi, rl)


def _sc_seg_body(nchunks, nper, msg, vj, z128, z16, sums_out, cnts_out,
                 vj_v, mbuf, ones_v, sums_sh, cnts_sh):
    c = lax.axis_index("c")
    s = lax.axis_index("s")
    wid = c * NS + s
    nw = NC * NS
    r0 = s * nper

    pltpu.sync_copy(z128.at[pl.ds(r0, nper)], sums_sh.at[pl.ds(r0, nper)])
    pltpu.sync_copy(z16.at[pl.ds(r0, nper)], cnts_sh.at[pl.ds(r0, nper)])

    def fill(i, carry):
        ones_v[i, :] = jnp.full((16,), 1.0, jnp.float32)
        return carry

    lax.fori_loop(0, CHUNK, fill, 0)
    plsc.subcore_barrier()

    def do_chunk(cid):
        off = cid * CHUNK
        pltpu.sync_copy(vj.at[pl.ds(off, CHUNK)], vj_v)
        pltpu.sync_copy(msg.at[pl.ds(off, CHUNK)], mbuf)
        pltpu.sync_copy(mbuf, sums_sh.at[vj_v], add=True)
        pltpu.sync_copy(ones_v, cnts_sh.at[vj_v], add=True)

    rounds = nchunks // nw
    rem = nchunks - rounds * nw

    def body(k, carry):
        do_chunk(wid + nw * k)
        return carry

    lax.fori_loop(0, rounds, body, 0)

    @pl.when(wid < rem)
    def _():
        do_chunk(rounds * nw + wid)

    plsc.subcore_barrier()
    pltpu.sync_copy(sums_sh.at[pl.ds(r0, nper)], sums_out.at[c, pl.ds(r0, nper)])
    pltpu.sync_copy(cnts_sh.at[pl.ds(r0, nper)], cnts_out.at[c, pl.ds(r0, nper)])


def _sc_segment(msg, vj, n):
    e, d = msg.shape
    assert n % NS == 0
    nper = n // NS
    nchunks = e // CHUNK
    z128 = jnp.zeros((n, d), jnp.float32)
    z16 = jnp.zeros((n, 16), jnp.float32)
    mesh = plsc.VectorSubcoreMesh(core_axis_name="c", subcore_axis_name="s")
    return pl.kernel(
        functools.partial(_sc_seg_body, nchunks, nper),
        out_type=[jax.ShapeDtypeStruct((NC, n, d), jnp.float32),
                  jax.ShapeDtypeStruct((NC, n, 16), jnp.float32)],
        mesh=mesh,
        scratch_types=[
            pltpu.VMEM((CHUNK,), jnp.int32),
            pltpu.VMEM((CHUNK, d), jnp.float32),
            pltpu.VMEM((CHUNK, 16), jnp.float32),
            pltpu.VMEM_SHARED((n, d), jnp.float32),
            pltpu.VMEM_SHARED((n, 16), jnp.float32),
        ],
    )(msg, vj, z128, z16)


def _tc_edge_body(hv, rv, ey, ws, fb, w, gb, out):
    eyv = ey[...]
    w12 = ws[1:2, :] + ws[2:3, :] * eyv
    pre = hv[...] * (ws[0:1, :] + rv[...] * w12) + fb[...]
    msg = jnp.tanh(pre)
    mm = jnp.dot(msg, w[...], preferred_element_type=jnp.float32)
    out[...] = msg + jnp.tanh(mm + gb[...])


def _tc_edge(hv, rv, ey, ws, fb, w, gb, block_e=2000):
    e, d = hv.shape
    assert e % block_e == 0
    grid = (e // block_e,)
    full = lambda shp: pl.BlockSpec(shp, lambda i: (0,) * len(shp))
    return pl.pallas_call(
        _tc_edge_body,
        grid=grid,
        in_specs=[
            pl.BlockSpec((block_e, d), lambda i: (i, 0)),
            pl.BlockSpec((block_e, d), lambda i: (i, 0)),
            pl.BlockSpec((block_e, 1), lambda i: (i, 0)),
            full((3, d)),
            full((1, d)),
            full((d, d)),
            full((1, d)),
        ],
        out_specs=pl.BlockSpec((block_e, d), lambda i: (i, 0)),
        out_shape=jax.ShapeDtypeStruct((e, d), jnp.float32),
    )(hv, rv, ey, ws, fb, w, gb)


def _tc_node_body(h, ps, pc, ent, fhw, fhb, gw, ghb, gruw, grub, out):
    d = h.shape[-1]
    ssum = ps[0] + ps[1]
    cnt = pc[0, :, 0:1] + pc[1, :, 0:1]
    aggr = ssum / jnp.maximum(cnt, 1.0)
    hh = h[...]
    e = ent[...]
    fw = fhw[...]
    u = (hh * (fw[0:1] + fw[1:2] * aggr + fw[2:3] * e)
         + aggr * (fw[3:4] + fw[5:6] * e) + fw[4:5] * e + fhb[...])
    upd = jnp.tanh(u)
    upd = upd + jnp.tanh(
        jnp.dot(upd, gw[...], preferred_element_type=jnp.float32) + ghb[...])
    zpre = (jnp.dot(hh, gruw[0:d, :], preferred_element_type=jnp.float32)
            + jnp.dot(upd, gruw[d:2 * d, :], preferred_element_type=jnp.float32)
            + grub[...])
    z = jax.nn.sigmoid(zpre)
    out[...] = (1.0 - z) * hh + z * upd


def _tc_node(h0, psums, pcnts, ent, fhw, fhb, gw, ghb, gruw, grub, block_n=2000):
    n, d = h0.shape
    assert n % block_n == 0
    grid = (n // block_n,)
    full = lambda shp: pl.BlockSpec(shp, lambda i: (0,) * len(shp))
    return pl.pallas_call(
        _tc_node_body,
        grid=grid,
        in_specs=[
            pl.BlockSpec((block_n, d), lambda i: (i, 0)),
            pl.BlockSpec((NC, block_n, d), lambda i: (0, i, 0)),
            pl.BlockSpec((NC, block_n, 16), lambda i: (0, i, 0)),
            pl.BlockSpec((block_n, d), lambda i: (i, 0)),
            full((6, d)),
            full((1, d)),
            full((d, d)),
            full((1, d)),
            full((2 * d, d)),
            full((1, d)),
        ],
        out_specs=pl.BlockSpec((block_n, d), lambda i: (i, 0)),
        out_shape=jax.ShapeDtypeStruct((n, d), jnp.float32),
    )(h0, psums, pcnts, ent, fhw, fhb, gw, ghb, gruw, grub)


def kernel(hidden, edges_y, selected_edges, ent_emb, rel_emb,
           f_msg_ws, f_msg_b, g_msg_W, g_msg_b,
           f_hid_ws, f_hid_b, g_hid_W, g_hid_b, gru_W, gru_b):
    _, n, d = hidden.shape
    e = selected_edges.shape[0]
    h0 = hidden[0]
    vi = selected_edges[:, 1]
    rl = selected_edges[:, 3]
    vj = selected_edges[:, 5]

    hv, rv = _sc_gather(h0, rel_emb, vi, rl)
    msg = _tc_edge(hv, rv, edges_y.reshape(e, 1), f_msg_ws,
                   f_msg_b.reshape(1, d), g_msg_W, g_msg_b.reshape(1, d))
    psums, pcnts = _sc_segment(msg, vj, n)
    out = _tc_node(h0, psums, pcnts, ent_emb, f_hid_ws,
                   f_hid_b.reshape(1, d), g_hid_W, g_hid_b.reshape(1, d),
                   gru_W, gru_b.reshape(1, d))
    return out.reshape(1, n, d)


# speedup vs baseline: 2.7385x; 2.7385x over previous
import functools

import jax
import jax.numpy as jnp
from jax import lax
from jax.experimental import pallas as pl
from jax.experimental.pallas import tpu as pltpu
from jax.experimental.pallas import tpu_sc as plsc

NC = 2
NS = 16
CHUNK = 128
RZ = 40


def _sc_gather_body(nchunks, h0, rel_t, vi, rl, hv_out, rv_out,
                    vi_v, rl_v, hv_v, rv_v, sem1, sem2):
    c = lax.axis_index("c")
    s = lax.axis_index("s")
    wid = c * NS + s
    nw = NC * NS

    def do_chunk(cid):
        off = cid * CHUNK
        pltpu.sync_copy(vi.at[pl.ds(off, CHUNK)], vi_v)
        pltpu.sync_copy(rl.at[pl.ds(off, CHUNK)], rl_v)
        cp1 = pltpu.async_copy(h0.at[vi_v], hv_v, sem1)
        cp2 = pltpu.async_copy(rel_t.at[rl_v], rv_v, sem2)
        cp1.wait()
        cp2.wait()
        pltpu.sync_copy(hv_v, hv_out.at[pl.ds(off, CHUNK)])
        pltpu.sync_copy(rv_v, rv_out.at[pl.ds(off, CHUNK)])

    rounds = nchunks // nw
    rem = nchunks - rounds * nw

    def body(k, carry):
        do_chunk(wid + nw * k)
        return carry

    lax.fori_loop(0, rounds, body, 0)

    @pl.when(wid < rem)
    def _():
        do_chunk(rounds * nw + wid)


def _sc_gather(h0, rel_emb, vi, rl):
    n, d = h0.shape
    e = vi.shape[0]
    assert e % CHUNK == 0
    nchunks = e // CHUNK
    mesh = plsc.VectorSubcoreMesh(core_axis_name="c", subcore_axis_name="s")
    return pl.kernel(
        functools.partial(_sc_gather_body, nchunks),
        out_type=[jax.ShapeDtypeStruct((e, d), jnp.float32),
                  jax.ShapeDtypeStruct((e, d), jnp.float32)],
        mesh=mesh,
        scratch_types=[
            pltpu.VMEM((CHUNK,), jnp.int32),
            pltpu.VMEM((CHUNK,), jnp.int32),
            pltpu.VMEM((CHUNK, d), jnp.float32),
            pltpu.VMEM((CHUNK, d), jnp.float32),
            pltpu.SemaphoreType.DMA,
            pltpu.SemaphoreType.DMA,
        ],
    )(h0, rel_emb, vi, rl)


NPT = 320


def _sc_seg_body(nchunks, msg, vj, sums_out, cnts_out,
                 vj_v, mbuf, probe, acc, acc16):
    c = lax.axis_index("c")
    s = lax.axis_index("s")
    wid = c * NS + s
    n0 = wid * NPT
    iota16 = lax.iota(jnp.int32, 16)

    def first_chunk_ge(target):
        def step(_, lohi):
            lo, hi = lohi
            done = lo >= hi
            mid = jnp.minimum((lo + hi) // 2, nchunks - 1)
            pltpu.sync_copy(vj.at[pl.ds(mid * CHUNK, 16)], probe)
            v = probe[...][0]
            cond = v >= target
            hi2 = jnp.where(done, hi, jnp.where(cond, mid, hi))
            lo2 = jnp.where(done, lo, jnp.where(cond, lo, mid + 1))
            return (lo2, hi2)

        nsteps = max(nchunks.bit_length(), 1)
        lo, _ = lax.fori_loop(0, nsteps, step,
                              (jnp.int32(0), jnp.int32(nchunks)))
        return lo

    c0 = jnp.maximum(first_chunk_ge(n0) - 1, 0)
    c1 = first_chunk_ge(n0 + NPT)

    def zacc(i, carry):
        for dc in range(8):
            acc[i, pl.ds(dc * 16, 16)] = jnp.zeros((16,), jnp.float32)
        acc16[i, :] = jnp.zeros((16,), jnp.float32)
        return carry

    lax.fori_loop(0, NPT, zacc, 0)
    ones16 = jnp.full((16,), 1.0, jnp.float32)

    def do_chunk(ci, carry):
        off = ci * CHUNK
        pltpu.sync_copy(vj.at[pl.ds(off, CHUNK)], vj_v.at[pl.ds(0, CHUNK)])
        pltpu.sync_copy(msg.at[pl.ds(off, CHUNK)], mbuf)

        def edge(e2, carry2):
            sv = vj_v[pl.ds(e2, 16)][0]
            rvb = jax.lax.broadcast(sv - n0, (16,))
            mask = (rvb >= 0) & (rvb < NPT)
            rvc = jnp.maximum(jnp.minimum(rvb, NPT - 1), 0)
            for dc in range(8):
                vals = mbuf[e2, pl.ds(dc * 16, 16)]
                plsc.addupdate_scatter(acc, [rvc, dc * 16 + iota16], vals,
                                       mask=mask)
            plsc.addupdate_scatter(acc16, [rvc, iota16], ones16, mask=mask)
            return carry2

        lax.fori_loop(0, CHUNK, edge, 0)
        return carry

    lax.fori_loop(c0, c1, do_chunk, 0)

    pltpu.sync_copy(acc, sums_out.at[pl.ds(n0, NPT)])
    pltpu.sync_copy(acc16, cnts_out.at[pl.ds(n0, NPT)])


def _sc_segment(msg, vj):
    e, d = msg.shape
    nchunks = e // CHUNK
    npad = NC * NS * NPT
    mesh = plsc.VectorSubcoreMesh(core_axis_name="c", subcore_axis_name="s")
    return pl.kernel(
        functools.partial(_sc_seg_body, nchunks),
        out_type=[jax.ShapeDtypeStruct((npad, d), jnp.float32),
                  jax.ShapeDtypeStruct((npad, 16), jnp.float32)],
        mesh=mesh,
        compiler_params=pltpu.CompilerParams(needs_layout_passes=False),
        scratch_types=[
            pltpu.VMEM((CHUNK + 16,), jnp.int32),
            pltpu.VMEM((CHUNK, d), jnp.float32),
            pltpu.VMEM((16,), jnp.int32),
            pltpu.VMEM((NPT, d), jnp.float32),
            pltpu.VMEM((NPT, 16), jnp.float32),
        ],
    )(msg, vj)


def _tc_edge_body(hv, rv, ey, ws, fb, w, gb, out):
    eyv = ey[...]
    w12 = ws[1:2, :] + ws[2:3, :] * eyv
    pre = hv[...] * (ws[0:1, :] + rv[...] * w12) + fb[...]
    msg = jnp.tanh(pre)
    mm = jnp.dot(msg, w[...], preferred_element_type=jnp.float32)
    out[...] = msg + jnp.tanh(mm + gb[...])


def _tc_edge(hv, rv, ey, ws, fb, w, gb, block_e=2000):
    e, d = hv.shape
    assert e % block_e == 0
    grid = (e // block_e,)
    full = lambda shp: pl.BlockSpec(shp, lambda i: (0,) * len(shp))
    return pl.pallas_call(
        _tc_edge_body,
        grid=grid,
        in_specs=[
            pl.BlockSpec((block_e, d), lambda i: (i, 0)),
            pl.BlockSpec((block_e, d), lambda i: (i, 0)),
            pl.BlockSpec((block_e, 1), lambda i: (i, 0)),
            full((3, d)),
            full((1, d)),
            full((d, d)),
            full((1, d)),
        ],
        out_specs=pl.BlockSpec((block_e, d), lambda i: (i, 0)),
        out_shape=jax.ShapeDtypeStruct((e, d), jnp.float32),
    )(hv, rv, ey, ws, fb, w, gb)


def _tc_node_body(h, ps, pc, ent, fhw, fhb, gw, ghb, gruw, grub, out):
    d = h.shape[-1]
    cnt = pc[:, 0:1]
    aggr = ps[...] / jnp.maximum(cnt, 1.0)
    hh = h[...]
    e = ent[...]
    fw = fhw[...]
    u = (hh * (fw[0:1] + fw[1:2] * aggr + fw[2:3] * e)
         + aggr * (fw[3:4] + fw[5:6] * e) + fw[4:5] * e + fhb[...])
    upd = jnp.tanh(u)
    upd = upd + jnp.tanh(
        jnp.dot(upd, gw[...], preferred_element_type=jnp.float32) + ghb[...])
    zpre = (jnp.dot(hh, gruw[0:d, :], preferred_element_type=jnp.float32)
            + jnp.dot(upd, gruw[d:2 * d, :], preferred_element_type=jnp.float32)
            + grub[...])
    z = jax.nn.sigmoid(zpre)
    out[...] = (1.0 - z) * hh + z * upd


def _tc_node(h0, psums, pcnts, ent, fhw, fhb, gw, ghb, gruw, grub, block_n=2000):
    n, d = h0.shape
    assert n % block_n == 0
    grid = (n // block_n,)
    full = lambda shp: pl.BlockSpec(shp, lambda i: (0,) * len(shp))
    return pl.pallas_call(
        _tc_node_body,
        grid=grid,
        in_specs=[
            pl.BlockSpec((block_n, d), lambda i: (i, 0)),
            pl.BlockSpec((block_n, d), lambda i: (i, 0)),
            pl.BlockSpec((block_n, 16), lambda i: (i, 0)),
            pl.BlockSpec((block_n, d), lambda i: (i, 0)),
            full((6, d)),
            full((1, d)),
            full((d, d)),
            full((1, d)),
            full((2 * d, d)),
            full((1, d)),
        ],
        out_specs=pl.BlockSpec((block_n, d), lambda i: (i, 0)),
        out_shape=jax.ShapeDtypeStruct((n, d), jnp.float32),
    )(h0, psums, pcnts, ent, fhw, fhb, gw, ghb, gruw, grub)


def kernel(hidden, edges_y, selected_edges, ent_emb, rel_emb,
           f_msg_ws, f_msg_b, g_msg_W, g_msg_b,
           f_hid_ws, f_hid_b, g_hid_W, g_hid_b, gru_W, gru_b):
    _, n, d = hidden.shape
    e = selected_edges.shape[0]
    h0 = hidden[0]
    vi = selected_edges[:, 1]
    rl = selected_edges[:, 3]
    vj = selected_edges[:, 5]

    hv, rv = _sc_gather(h0, rel_emb, vi, rl)
    msg = _tc_edge(hv, rv, edges_y.reshape(e, 1), f_msg_ws,
                   f_msg_b.reshape(1, d), g_msg_W, g_msg_b.reshape(1, d))
    sums_p, cnts_p = _sc_segment(msg, vj)
    out = _tc_node(h0, sums_p[:n], cnts_p[:n], ent_emb, f_hid_ws,
                   f_hid_b.reshape(1, d), g_hid_W, g_hid_b.reshape(1, d),
                   gru_W, gru_b.reshape(1, d))
    return out.reshape(1, n, d)


# speedup vs baseline: 3.5947x; 1.3126x over previous
import functools

import jax
import jax.numpy as jnp
from jax import lax
from jax.experimental import pallas as pl
from jax.experimental.pallas import tpu as pltpu
from jax.experimental.pallas import tpu_sc as plsc

NC = 2
NS = 16
CHUNK = 128
RZ = 40


def _sc_gather_body(nchunks, h0, rel_t, vi, rl, hv_out, rv_out,
                    vi_v0, rl_v0, hv_v0, rv_v0, vi_v1, rl_v1, hv_v1, rv_v1,
                    gs0, rs0, gs1, rs1):
    c = lax.axis_index("c")
    s = lax.axis_index("s")
    wid = c * NS + s
    nw = NC * NS
    rounds = nchunks // nw
    rem = nchunks - rounds * nw
    bufs = ((vi_v0, rl_v0, hv_v0, rv_v0, gs0, rs0),
            (vi_v1, rl_v1, hv_v1, rv_v1, gs1, rs1))

    def start_gather(b, k):
        viv, rlv, hvv, rvv, gs, rs = b
        off = (wid + nw * k) * CHUNK
        pltpu.sync_copy(vi.at[pl.ds(off, CHUNK)], viv)
        pltpu.sync_copy(rl.at[pl.ds(off, CHUNK)], rlv)
        pltpu.async_copy(h0.at[viv], hvv, gs)
        pltpu.async_copy(rel_t.at[rlv], rvv, rs)

    def wait_gather(b):
        viv, rlv, hvv, rvv, gs, rs = b
        pltpu.make_async_copy(h0.at[viv], hvv, gs).wait()
        pltpu.make_async_copy(rel_t.at[rlv], rvv, rs).wait()

    def write_out(b, k):
        viv, rlv, hvv, rvv, gs, rs = b
        off = (wid + nw * k) * CHUNK
        pltpu.sync_copy(hvv, hv_out.at[pl.ds(off, CHUNK)])
        pltpu.sync_copy(rvv, rv_out.at[pl.ds(off, CHUNK)])

    start_gather(bufs[0], 0)
    npairs = rounds // 2

    def pbody(j, carry):
        k0 = 2 * j
        wait_gather(bufs[0])
        start_gather(bufs[1], k0 + 1)
        write_out(bufs[0], k0)
        wait_gather(bufs[1])

        @pl.when(k0 + 2 < rounds)
        def _():
            start_gather(bufs[0], k0 + 2)

        write_out(bufs[1], k0 + 1)
        return carry

    lax.fori_loop(0, npairs, pbody, 0)
    if rounds % 2 == 1:
        wait_gather(bufs[0])
        write_out(bufs[0], rounds - 1)

    @pl.when(wid < rem)
    def _():
        b = bufs[0]
        viv, rlv, hvv, rvv, gs, rs = b
        off = (rounds * nw + wid) * CHUNK
        pltpu.sync_copy(vi.at[pl.ds(off, CHUNK)], viv)
        pltpu.sync_copy(rl.at[pl.ds(off, CHUNK)], rlv)
        pltpu.async_copy(h0.at[viv], hvv, gs)
        pltpu.async_copy(rel_t.at[rlv], rvv, rs)
        pltpu.make_async_copy(h0.at[viv], hvv, gs).wait()
        pltpu.make_async_copy(rel_t.at[rlv], rvv, rs).wait()
        pltpu.sync_copy(hvv, hv_out.at[pl.ds(off, CHUNK)])
        pltpu.sync_copy(rvv, rv_out.at[pl.ds(off, CHUNK)])


def _sc_gather(h0, rel_emb, vi, rl):
    n, d = h0.shape
    e = vi.shape[0]
    assert e % CHUNK == 0
    nchunks = e // CHUNK
    mesh = plsc.VectorSubcoreMesh(core_axis_name="c", subcore_axis_name="s")
    return pl.kernel(
        functools.partial(_sc_gather_body, nchunks),
        out_type=[jax.ShapeDtypeStruct((e, d), jnp.float32),
                  jax.ShapeDtypeStruct((e, d), jnp.float32)],
        mesh=mesh,
        scratch_types=[
            pltpu.VMEM((CHUNK,), jnp.int32),
            pltpu.VMEM((CHUNK,), jnp.int32),
            pltpu.VMEM((CHUNK, d), jnp.float32),
            pltpu.VMEM((CHUNK, d), jnp.float32),
            pltpu.VMEM((CHUNK,), jnp.int32),
            pltpu.VMEM((CHUNK,), jnp.int32),
            pltpu.VMEM((CHUNK, d), jnp.float32),
            pltpu.VMEM((CHUNK, d), jnp.float32),
            pltpu.SemaphoreType.DMA,
            pltpu.SemaphoreType.DMA,
            pltpu.SemaphoreType.DMA,
            pltpu.SemaphoreType.DMA,
        ],
    )(h0, rel_emb, vi, rl)


NPT = 320


def _sc_seg_body(nchunks, msg, vj, sums_out, cnts_out,
                 vj_v, mbuf, probe, acc, acc16):
    c = lax.axis_index("c")
    s = lax.axis_index("s")
    wid = c * NS + s
    n0 = wid * NPT
    iota16 = lax.iota(jnp.int32, 16)

    def first_chunk_ge(target):
        def step(_, lohi):
            lo, hi = lohi
            done = lo >= hi
            mid = jnp.minimum((lo + hi) // 2, nchunks - 1)
            pltpu.sync_copy(vj.at[pl.ds(mid * CHUNK, 16)], probe)
            v = probe[...][0]
            cond = v >= target
            hi2 = jnp.where(done, hi, jnp.where(cond, mid, hi))
            lo2 = jnp.where(done, lo, jnp.where(cond, lo, mid + 1))
            return (lo2, hi2)

        nsteps = max(nchunks.bit_length(), 1)
        lo, _ = lax.fori_loop(0, nsteps, step,
                              (jnp.int32(0), jnp.int32(nchunks)))
        return lo

    c0 = jnp.maximum(first_chunk_ge(n0) - 1, 0)
    c1 = first_chunk_ge(n0 + NPT)

    @plsc.parallel_loop(0, NPT, unroll=4)
    def zacc(i):
        for dc in range(8):
            acc[i, pl.ds(dc * 16, 16)] = jnp.zeros((16,), jnp.float32)
        acc16[i, :] = jnp.zeros((16,), jnp.float32)
    ones16 = jnp.full((16,), 1.0, jnp.float32)

    def do_chunk(ci, carry):
        off = ci * CHUNK
        pltpu.sync_copy(vj.at[pl.ds(off, CHUNK)], vj_v.at[pl.ds(0, CHUNK)])
        pltpu.sync_copy(msg.at[pl.ds(off, CHUNK)], mbuf)

        @plsc.parallel_loop(0, CHUNK, unroll=8)
        def edge(e2):
            sv = vj_v[pl.ds(e2, 16)][0]
            rvb = jax.lax.broadcast(sv - n0, (16,))
            mask = (rvb >= 0) & (rvb < NPT)
            rvc = jnp.maximum(jnp.minimum(rvb, NPT - 1), 0)
            for dc in range(8):
                vals = mbuf[e2, pl.ds(dc * 16, 16)]
                plsc.addupdate_scatter(acc, [rvc, dc * 16 + iota16], vals,
                                       mask=mask)
            plsc.addupdate_scatter(acc16, [rvc, iota16], ones16, mask=mask)

        return carry

    lax.fori_loop(c0, c1, do_chunk, 0)

    pltpu.sync_copy(acc, sums_out.at[pl.ds(n0, NPT)])
    pltpu.sync_copy(acc16, cnts_out.at[pl.ds(n0, NPT)])


def _sc_segment(msg, vj):
    e, d = msg.shape
    nchunks = e // CHUNK
    npad = NC * NS * NPT
    mesh = plsc.VectorSubcoreMesh(core_axis_name="c", subcore_axis_name="s")
    return pl.kernel(
        functools.partial(_sc_seg_body, nchunks),
        out_type=[jax.ShapeDtypeStruct((npad, d), jnp.float32),
                  jax.ShapeDtypeStruct((npad, 16), jnp.float32)],
        mesh=mesh,
        compiler_params=pltpu.CompilerParams(needs_layout_passes=False),
        scratch_types=[
            pltpu.VMEM((CHUNK + 16,), jnp.int32),
            pltpu.VMEM((CHUNK, d), jnp.float32),
            pltpu.VMEM((16,), jnp.int32),
            pltpu.VMEM((NPT, d), jnp.float32),
            pltpu.VMEM((NPT, 16), jnp.float32),
        ],
    )(msg, vj)


def _tc_edge_body(hv, rv, ey, ws, fb, w, gb, out):
    eyv = ey[...]
    w12 = ws[1:2, :] + ws[2:3, :] * eyv
    pre = hv[...] * (ws[0:1, :] + rv[...] * w12) + fb[...]
    msg = jnp.tanh(pre)
    mm = jnp.dot(msg, w[...], preferred_element_type=jnp.float32)
    out[...] = msg + jnp.tanh(mm + gb[...])


def _tc_edge(hv, rv, ey, ws, fb, w, gb, block_e=2000):
    e, d = hv.shape
    assert e % block_e == 0
    grid = (e // block_e,)
    full = lambda shp: pl.BlockSpec(shp, lambda i: (0,) * len(shp))
    return pl.pallas_call(
        _tc_edge_body,
        grid=grid,
        in_specs=[
            pl.BlockSpec((block_e, d), lambda i: (i, 0)),
            pl.BlockSpec((block_e, d), lambda i: (i, 0)),
            pl.BlockSpec((block_e, 1), lambda i: (i, 0)),
            full((3, d)),
            full((1, d)),
            full((d, d)),
            full((1, d)),
        ],
        out_specs=pl.BlockSpec((block_e, d), lambda i: (i, 0)),
        out_shape=jax.ShapeDtypeStruct((e, d), jnp.float32),
    )(hv, rv, ey, ws, fb, w, gb)


def _tc_node_body(h, ps, pc, ent, fhw, fhb, gw, ghb, gruw, grub, out):
    d = h.shape[-1]
    cnt = pc[:, 0:1]
    aggr = ps[...] / jnp.maximum(cnt, 1.0)
    hh = h[...]
    e = ent[...]
    fw = fhw[...]
    u = (hh * (fw[0:1] + fw[1:2] * aggr + fw[2:3] * e)
         + aggr * (fw[3:4] + fw[5:6] * e) + fw[4:5] * e + fhb[...])
    upd = jnp.tanh(u)
    upd = upd + jnp.tanh(
        jnp.dot(upd, gw[...], preferred_element_type=jnp.float32) + ghb[...])
    zpre = (jnp.dot(hh, gruw[0:d, :], preferred_element_type=jnp.float32)
            + jnp.dot(upd, gruw[d:2 * d, :], preferred_element_type=jnp.float32)
            + grub[...])
    z = jax.nn.sigmoid(zpre)
    out[...] = (1.0 - z) * hh + z * upd


def _tc_node(h0, psums, pcnts, ent, fhw, fhb, gw, ghb, gruw, grub, block_n=2000):
    n, d = h0.shape
    assert n % block_n == 0
    grid = (n // block_n,)
    full = lambda shp: pl.BlockSpec(shp, lambda i: (0,) * len(shp))
    return pl.pallas_call(
        _tc_node_body,
        grid=grid,
        in_specs=[
            pl.BlockSpec((block_n, d), lambda i: (i, 0)),
            pl.BlockSpec((block_n, d), lambda i: (i, 0)),
            pl.BlockSpec((block_n, 16), lambda i: (i, 0)),
            pl.BlockSpec((block_n, d), lambda i: (i, 0)),
            full((6, d)),
            full((1, d)),
            full((d, d)),
            full((1, d)),
            full((2 * d, d)),
            full((1, d)),
        ],
        out_specs=pl.BlockSpec((block_n, d), lambda i: (i, 0)),
        out_shape=jax.ShapeDtypeStruct((n, d), jnp.float32),
    )(h0, psums, pcnts, ent, fhw, fhb, gw, ghb, gruw, grub)


def kernel(hidden, edges_y, selected_edges, ent_emb, rel_emb,
           f_msg_ws, f_msg_b, g_msg_W, g_msg_b,
           f_hid_ws, f_hid_b, g_hid_W, g_hid_b, gru_W, gru_b):
    _, n, d = hidden.shape
    e = selected_edges.shape[0]
    h0 = hidden[0]
    vi = selected_edges[:, 1]
    rl = selected_edges[:, 3]
    vj = selected_edges[:, 5]

    hv, rv = _sc_gather(h0, rel_emb, vi, rl)
    msg = _tc_edge(hv, rv, edges_y.reshape(e, 1), f_msg_ws,
                   f_msg_b.reshape(1, d), g_msg_W, g_msg_b.reshape(1, d))
    sums_p, cnts_p = _sc_segment(msg, vj)
    out = _tc_node(h0, sums_p[:n], cnts_p[:n], ent_emb, f_hid_ws,
                   f_hid_b.reshape(1, d), g_hid_W, g_hid_b.reshape(1, d),
                   gru_W, gru_b.reshape(1, d))
    return out.reshape(1, n, d)


# speedup vs baseline: 3.6025x; 1.0022x over previous
import functools

import jax
import jax.numpy as jnp
from jax import lax
from jax.experimental import pallas as pl
from jax.experimental.pallas import tpu as pltpu
from jax.experimental.pallas import tpu_sc as plsc

NC = 2
NS = 16
CHUNK = 128
RZ = 40


def _sc_gather_body(nchunks, h0, rel_t, vi, rl, hv_out, rv_out,
                    vi_t, rl_t_v, vi_r, rl_r, hv_v0, rv_v0, hv_v1, rv_v1,
                    gs0, rs0, gs1, rs1):
    c = lax.axis_index("c")
    s = lax.axis_index("s")
    wid = c * NS + s
    nw = NC * NS
    rounds = nchunks // nw
    rem = nchunks - rounds * nw
    tbase = wid * rounds * CHUNK

    pltpu.sync_copy(vi.at[pl.ds(tbase, rounds * CHUNK)], vi_t)
    pltpu.sync_copy(rl.at[pl.ds(tbase, rounds * CHUNK)], rl_t_v)

    bufs = ((hv_v0, rv_v0, gs0, rs0), (hv_v1, rv_v1, gs1, rs1))

    def idx_slice(k):
        return (vi_t.at[pl.ds(k * CHUNK, CHUNK)],
                rl_t_v.at[pl.ds(k * CHUNK, CHUNK)])

    def start_gather(b, k):
        hvv, rvv, gs, rs = b
        vs, rsl = idx_slice(k)
        pltpu.async_copy(h0.at[vs], hvv, gs)
        pltpu.async_copy(rel_t.at[rsl], rvv, rs)

    def wait_gather(b, k):
        hvv, rvv, gs, rs = b
        vs, rsl = idx_slice(k)
        pltpu.make_async_copy(h0.at[vs], hvv, gs).wait()
        pltpu.make_async_copy(rel_t.at[rsl], rvv, rs).wait()

    def write_out(b, k):
        hvv, rvv, gs, rs = b
        off = tbase + k * CHUNK
        pltpu.sync_copy(hvv, hv_out.at[pl.ds(off, CHUNK)])
        pltpu.sync_copy(rvv, rv_out.at[pl.ds(off, CHUNK)])

    start_gather(bufs[0], 0)
    npairs = rounds // 2

    def pbody(j, carry):
        k0 = 2 * j
        wait_gather(bufs[0], k0)
        start_gather(bufs[1], k0 + 1)
        write_out(bufs[0], k0)
        wait_gather(bufs[1], k0 + 1)

        @pl.when(k0 + 2 < rounds)
        def _():
            start_gather(bufs[0], k0 + 2)

        write_out(bufs[1], k0 + 1)
        return carry

    lax.fori_loop(0, npairs, pbody, 0)
    if rounds % 2 == 1:
        wait_gather(bufs[0], rounds - 1)
        write_out(bufs[0], rounds - 1)

    @pl.when(wid < rem)
    def _():
        hvv, rvv, gs, rs = bufs[0]
        off = (rounds * nw + wid) * CHUNK
        pltpu.sync_copy(vi.at[pl.ds(off, CHUNK)], vi_r)
        pltpu.sync_copy(rl.at[pl.ds(off, CHUNK)], rl_r)
        pltpu.async_copy(h0.at[vi_r], hvv, gs)
        pltpu.async_copy(rel_t.at[rl_r], rvv, rs)
        pltpu.make_async_copy(h0.at[vi_r], hvv, gs).wait()
        pltpu.make_async_copy(rel_t.at[rl_r], rvv, rs).wait()
        pltpu.sync_copy(hvv, hv_out.at[pl.ds(off, CHUNK)])
        pltpu.sync_copy(rvv, rv_out.at[pl.ds(off, CHUNK)])


def _sc_gather(h0, rel_emb, vi, rl):
    n, d = h0.shape
    e = vi.shape[0]
    assert e % CHUNK == 0
    nchunks = e // CHUNK
    rounds = nchunks // (NC * NS)
    mesh = plsc.VectorSubcoreMesh(core_axis_name="c", subcore_axis_name="s")
    return pl.kernel(
        functools.partial(_sc_gather_body, nchunks),
        out_type=[jax.ShapeDtypeStruct((e, d), jnp.float32),
                  jax.ShapeDtypeStruct((e, d), jnp.float32)],
        mesh=mesh,
        scratch_types=[
            pltpu.VMEM((rounds * CHUNK,), jnp.int32),
            pltpu.VMEM((rounds * CHUNK,), jnp.int32),
            pltpu.VMEM((CHUNK,), jnp.int32),
            pltpu.VMEM((CHUNK,), jnp.int32),
            pltpu.VMEM((CHUNK, d), jnp.float32),
            pltpu.VMEM((CHUNK, d), jnp.float32),
            pltpu.VMEM((CHUNK, d), jnp.float32),
            pltpu.VMEM((CHUNK, d), jnp.float32),
            pltpu.SemaphoreType.DMA,
            pltpu.SemaphoreType.DMA,
            pltpu.SemaphoreType.DMA,
            pltpu.SemaphoreType.DMA,
        ],
    )(h0, rel_emb, vi, rl)


NPT = 320


def _sc_seg_body(nchunks, msg, vj, sums_out, cnts_out,
                 vj_v, mbuf, probe, acc, acc16):
    c = lax.axis_index("c")
    s = lax.axis_index("s")
    wid = c * NS + s
    n0 = wid * NPT
    iota16 = lax.iota(jnp.int32, 16)

    def first_chunk_ge(target):
        def step(_, lohi):
            lo, hi = lohi
            done = lo >= hi
            mid = jnp.minimum((lo + hi) // 2, nchunks - 1)
            pltpu.sync_copy(vj.at[pl.ds(mid * CHUNK, 16)], probe)
            v = probe[...][0]
            cond = v >= target
            hi2 = jnp.where(done, hi, jnp.where(cond, mid, hi))
            lo2 = jnp.where(done, lo, jnp.where(cond, lo, mid + 1))
            return (lo2, hi2)

        nsteps = max(nchunks.bit_length(), 1)
        lo, _ = lax.fori_loop(0, nsteps, step,
                              (jnp.int32(0), jnp.int32(nchunks)))
        return lo

    c0 = jnp.maximum(first_chunk_ge(n0) - 1, 0)
    c1 = first_chunk_ge(n0 + NPT)

    @plsc.parallel_loop(0, NPT, unroll=4)
    def zacc(i):
        for dc in range(8):
            acc[i, pl.ds(dc * 16, 16)] = jnp.zeros((16,), jnp.float32)
        acc16[i, :] = jnp.zeros((16,), jnp.float32)
    ones16 = jnp.full((16,), 1.0, jnp.float32)

    def do_chunk(ci, carry):
        off = ci * CHUNK
        pltpu.sync_copy(vj.at[pl.ds(off, CHUNK)], vj_v.at[pl.ds(0, CHUNK)])
        pltpu.sync_copy(msg.at[pl.ds(off, CHUNK)], mbuf)

        @plsc.parallel_loop(0, CHUNK, unroll=8)
        def edge(e2):
            sv = vj_v[pl.ds(e2, 16)][0]
            rvb = jax.lax.broadcast(sv - n0, (16,))
            mask = (rvb >= 0) & (rvb < NPT)
            rvc = jnp.maximum(jnp.minimum(rvb, NPT - 1), 0)
            for dc in range(8):
                vals = mbuf[e2, pl.ds(dc * 16, 16)]
                plsc.addupdate_scatter(acc, [rvc, dc * 16 + iota16], vals,
                                       mask=mask)
            plsc.addupdate_scatter(acc16, [rvc, iota16], ones16, mask=mask)

        return carry

    lax.fori_loop(c0, c1, do_chunk, 0)

    pltpu.sync_copy(acc, sums_out.at[pl.ds(n0, NPT)])
    pltpu.sync_copy(acc16, cnts_out.at[pl.ds(n0, NPT)])


def _sc_segment(msg, vj):
    e, d = msg.shape
    nchunks = e // CHUNK
    npad = NC * NS * NPT
    mesh = plsc.VectorSubcoreMesh(core_axis_name="c", subcore_axis_name="s")
    return pl.kernel(
        functools.partial(_sc_seg_body, nchunks),
        out_type=[jax.ShapeDtypeStruct((npad, d), jnp.float32),
                  jax.ShapeDtypeStruct((npad, 16), jnp.float32)],
        mesh=mesh,
        compiler_params=pltpu.CompilerParams(needs_layout_passes=False),
        scratch_types=[
            pltpu.VMEM((CHUNK + 16,), jnp.int32),
            pltpu.VMEM((CHUNK, d), jnp.float32),
            pltpu.VMEM((16,), jnp.int32),
            pltpu.VMEM((NPT, d), jnp.float32),
            pltpu.VMEM((NPT, 16), jnp.float32),
        ],
    )(msg, vj)


def _tc_edge_body(hv, rv, ey, ws, fb, w, gb, out):
    eyv = ey[...]
    w12 = ws[1:2, :] + ws[2:3, :] * eyv
    pre = hv[...] * (ws[0:1, :] + rv[...] * w12) + fb[...]
    msg = jnp.tanh(pre)
    mm = jnp.dot(msg, w[...], preferred_element_type=jnp.float32)
    out[...] = msg + jnp.tanh(mm + gb[...])


def _tc_edge(hv, rv, ey, ws, fb, w, gb, block_e=2000):
    e, d = hv.shape
    assert e % block_e == 0
    grid = (e // block_e,)
    full = lambda shp: pl.BlockSpec(shp, lambda i: (0,) * len(shp))
    return pl.pallas_call(
        _tc_edge_body,
        grid=grid,
        in_specs=[
            pl.BlockSpec((block_e, d), lambda i: (i, 0)),
            pl.BlockSpec((block_e, d), lambda i: (i, 0)),
            pl.BlockSpec((block_e, 1), lambda i: (i, 0)),
            full((3, d)),
            full((1, d)),
            full((d, d)),
            full((1, d)),
        ],
        out_specs=pl.BlockSpec((block_e, d), lambda i: (i, 0)),
        out_shape=jax.ShapeDtypeStruct((e, d), jnp.float32),
    )(hv, rv, ey, ws, fb, w, gb)


def _tc_node_body(h, ps, pc, ent, fhw, fhb, gw, ghb, gruw, grub, out):
    d = h.shape[-1]
    cnt = pc[:, 0:1]
    aggr = ps[...] / jnp.maximum(cnt, 1.0)
    hh = h[...]
    e = ent[...]
    fw = fhw[...]
    u = (hh * (fw[0:1] + fw[1:2] * aggr + fw[2:3] * e)
         + aggr * (fw[3:4] + fw[5:6] * e) + fw[4:5] * e + fhb[...])
    upd = jnp.tanh(u)
    upd = upd + jnp.tanh(
        jnp.dot(upd, gw[...], preferred_element_type=jnp.float32) + ghb[...])
    zpre = (jnp.dot(hh, gruw[0:d, :], preferred_element_type=jnp.float32)
            + jnp.dot(upd, gruw[d:2 * d, :], preferred_element_type=jnp.float32)
            + grub[...])
    z = jax.nn.sigmoid(zpre)
    out[...] = (1.0 - z) * hh + z * upd


def _tc_node(h0, psums, pcnts, ent, fhw, fhb, gw, ghb, gruw, grub, block_n=2000):
    n, d = h0.shape
    assert n % block_n == 0
    grid = (n // block_n,)
    full = lambda shp: pl.BlockSpec(shp, lambda i: (0,) * len(shp))
    return pl.pallas_call(
        _tc_node_body,
        grid=grid,
        in_specs=[
            pl.BlockSpec((block_n, d), lambda i: (i, 0)),
            pl.BlockSpec((block_n, d), lambda i: (i, 0)),
            pl.BlockSpec((block_n, 16), lambda i: (i, 0)),
            pl.BlockSpec((block_n, d), lambda i: (i, 0)),
            full((6, d)),
            full((1, d)),
            full((d, d)),
            full((1, d)),
            full((2 * d, d)),
            full((1, d)),
        ],
        out_specs=pl.BlockSpec((block_n, d), lambda i: (i, 0)),
        out_shape=jax.ShapeDtypeStruct((n, d), jnp.float32),
    )(h0, psums, pcnts, ent, fhw, fhb, gw, ghb, gruw, grub)


def kernel(hidden, edges_y, selected_edges, ent_emb, rel_emb,
           f_msg_ws, f_msg_b, g_msg_W, g_msg_b,
           f_hid_ws, f_hid_b, g_hid_W, g_hid_b, gru_W, gru_b):
    _, n, d = hidden.shape
    e = selected_edges.shape[0]
    h0 = hidden[0]
    vi = selected_edges[:, 1]
    rl = selected_edges[:, 3]
    vj = selected_edges[:, 5]

    hv, rv = _sc_gather(h0, rel_emb, vi, rl)
    msg = _tc_edge(hv, rv, edges_y.reshape(e, 1), f_msg_ws,
                   f_msg_b.reshape(1, d), g_msg_W, g_msg_b.reshape(1, d))
    sums_p, cnts_p = _sc_segment(msg, vj)
    out = _tc_node(h0, sums_p[:n], cnts_p[:n], ent_emb, f_hid_ws,
                   f_hid_b.reshape(1, d), g_hid_W, g_hid_b.reshape(1, d),
                   gru_W, gru_b.reshape(1, d))
    return out.reshape(1, n, d)


# speedup vs baseline: 4.2988x; 1.1933x over previous
import functools

import jax
import jax.numpy as jnp
from jax import lax
from jax.experimental import pallas as pl
from jax.experimental.pallas import tpu as pltpu
from jax.experimental.pallas import tpu_sc as plsc

NC = 2
NS = 16
CHUNK = 128
RZ = 40


def _sc_gather_body(nchunks, h0, vi, hv_out,
                    vi_t, vi_r, hv_v0, hv_v1, gs0, gs1):
    c = lax.axis_index("c")
    s = lax.axis_index("s")
    wid = c * NS + s
    nw = NC * NS
    rounds = nchunks // nw
    rem = nchunks - rounds * nw
    tbase = wid * rounds * CHUNK

    pltpu.sync_copy(vi.at[pl.ds(tbase, rounds * CHUNK)], vi_t)
    bufs = ((hv_v0, gs0), (hv_v1, gs1))

    def idx_slice(k):
        return vi_t.at[pl.ds(k * CHUNK, CHUNK)]

    def start_gather(b, k):
        hvv, gs = b
        pltpu.async_copy(h0.at[idx_slice(k)], hvv, gs)

    def wait_gather(b, k):
        hvv, gs = b
        pltpu.make_async_copy(h0.at[idx_slice(k)], hvv, gs).wait()

    def write_out(b, k):
        hvv, gs = b
        off = tbase + k * CHUNK
        pltpu.sync_copy(hvv, hv_out.at[pl.ds(off, CHUNK)])

    start_gather(bufs[0], 0)
    npairs = rounds // 2

    def pbody(j, carry):
        k0 = 2 * j
        wait_gather(bufs[0], k0)
        start_gather(bufs[1], k0 + 1)
        write_out(bufs[0], k0)
        wait_gather(bufs[1], k0 + 1)

        @pl.when(k0 + 2 < rounds)
        def _():
            start_gather(bufs[0], k0 + 2)

        write_out(bufs[1], k0 + 1)
        return carry

    lax.fori_loop(0, npairs, pbody, 0)
    if rounds % 2 == 1:
        wait_gather(bufs[0], rounds - 1)
        write_out(bufs[0], rounds - 1)

    @pl.when(wid < rem)
    def _():
        hvv, gs = bufs[0]
        off = (rounds * nw + wid) * CHUNK
        pltpu.sync_copy(vi.at[pl.ds(off, CHUNK)], vi_r)
        pltpu.async_copy(h0.at[vi_r], hvv, gs)
        pltpu.make_async_copy(h0.at[vi_r], hvv, gs).wait()
        pltpu.sync_copy(hvv, hv_out.at[pl.ds(off, CHUNK)])


def _sc_gather(h0, vi):
    n, d = h0.shape
    e = vi.shape[0]
    assert e % CHUNK == 0
    nchunks = e // CHUNK
    rounds = nchunks // (NC * NS)
    mesh = plsc.VectorSubcoreMesh(core_axis_name="c", subcore_axis_name="s")
    return pl.kernel(
        functools.partial(_sc_gather_body, nchunks),
        out_type=jax.ShapeDtypeStruct((e, d), jnp.float32),
        mesh=mesh,
        scratch_types=[
            pltpu.VMEM((rounds * CHUNK,), jnp.int32),
            pltpu.VMEM((CHUNK,), jnp.int32),
            pltpu.VMEM((CHUNK, d), jnp.float32),
            pltpu.VMEM((CHUNK, d), jnp.float32),
            pltpu.SemaphoreType.DMA,
            pltpu.SemaphoreType.DMA,
        ],
    )(h0, vi)


NPT = 320


def _sc_seg_body(nchunks, msg, vj, sums_out, cnts_out,
                 vj_v, mbuf, probe, acc, acc16):
    c = lax.axis_index("c")
    s = lax.axis_index("s")
    wid = c * NS + s
    n0 = wid * NPT
    iota16 = lax.iota(jnp.int32, 16)

    def first_chunk_ge(target):
        def step(_, lohi):
            lo, hi = lohi
            done = lo >= hi
            mid = jnp.minimum((lo + hi) // 2, nchunks - 1)
            pltpu.sync_copy(vj.at[pl.ds(mid * CHUNK, 16)], probe)
            v = probe[...][0]
            cond = v >= target
            hi2 = jnp.where(done, hi, jnp.where(cond, mid, hi))
            lo2 = jnp.where(done, lo, jnp.where(cond, lo, mid + 1))
            return (lo2, hi2)

        nsteps = max(nchunks.bit_length(), 1)
        lo, _ = lax.fori_loop(0, nsteps, step,
                              (jnp.int32(0), jnp.int32(nchunks)))
        return lo

    c0 = jnp.maximum(first_chunk_ge(n0) - 1, 0)
    c1 = first_chunk_ge(n0 + NPT)

    @plsc.parallel_loop(0, NPT, unroll=4)
    def zacc(i):
        for dc in range(8):
            acc[i, pl.ds(dc * 16, 16)] = jnp.zeros((16,), jnp.float32)
        acc16[i, :] = jnp.zeros((16,), jnp.float32)
    ones16 = jnp.full((16,), 1.0, jnp.float32)

    def do_chunk(ci, carry):
        off = ci * CHUNK
        pltpu.sync_copy(vj.at[pl.ds(off, CHUNK)], vj_v.at[pl.ds(0, CHUNK)])
        pltpu.sync_copy(msg.at[pl.ds(off, CHUNK)], mbuf)

        @plsc.parallel_loop(0, CHUNK, unroll=8)
        def edge(e2):
            sv = vj_v[pl.ds(e2, 16)][0]
            rvb = jax.lax.broadcast(sv - n0, (16,))
            mask = (rvb >= 0) & (rvb < NPT)
            rvc = jnp.maximum(jnp.minimum(rvb, NPT - 1), 0)
            for dc in range(8):
                vals = mbuf[e2, pl.ds(dc * 16, 16)]
                plsc.addupdate_scatter(acc, [rvc, dc * 16 + iota16], vals,
                                       mask=mask)
            plsc.addupdate_scatter(acc16, [rvc, iota16], ones16, mask=mask)

        return carry

    lax.fori_loop(c0, c1, do_chunk, 0)

    pltpu.sync_copy(acc, sums_out.at[pl.ds(n0, NPT)])
    pltpu.sync_copy(acc16, cnts_out.at[pl.ds(n0, NPT)])


def _sc_segment(msg, vj):
    e, d = msg.shape
    nchunks = e // CHUNK
    npad = NC * NS * NPT
    mesh = plsc.VectorSubcoreMesh(core_axis_name="c", subcore_axis_name="s")
    return pl.kernel(
        functools.partial(_sc_seg_body, nchunks),
        out_type=[jax.ShapeDtypeStruct((npad, d), jnp.float32),
                  jax.ShapeDtypeStruct((npad, 16), jnp.float32)],
        mesh=mesh,
        compiler_params=pltpu.CompilerParams(needs_layout_passes=False),
        scratch_types=[
            pltpu.VMEM((CHUNK + 16,), jnp.int32),
            pltpu.VMEM((CHUNK, d), jnp.float32),
            pltpu.VMEM((16,), jnp.int32),
            pltpu.VMEM((NPT, d), jnp.float32),
            pltpu.VMEM((NPT, 16), jnp.float32),
        ],
    )(msg, vj)


def _tc_edge_body(hv, rel_b, ey, remb, ws, fb, w, gb, out):
    nrel = remb.shape[0]
    be = hv.shape[0]
    r_ids = rel_b[...]
    riota = jax.lax.broadcasted_iota(jnp.int32, (be, nrel), 1)
    onehot = (riota == r_ids).astype(jnp.float32)
    rv = jnp.dot(onehot, remb[...], preferred_element_type=jnp.float32)
    eyv = ey[...]
    w12 = ws[1:2, :] + ws[2:3, :] * eyv
    pre = hv[...] * (ws[0:1, :] + rv * w12) + fb[...]
    msg = jnp.tanh(pre)
    mm = jnp.dot(msg, w[...], preferred_element_type=jnp.float32)
    out[...] = msg + jnp.tanh(mm + gb[...])


def _tc_edge(hv, rel_b, ey, remb, ws, fb, w, gb, block_e=2000):
    e, d = hv.shape
    nrel = remb.shape[0]
    assert e % block_e == 0
    grid = (e // block_e,)
    full = lambda shp: pl.BlockSpec(shp, lambda i: (0,) * len(shp))
    return pl.pallas_call(
        _tc_edge_body,
        grid=grid,
        in_specs=[
            pl.BlockSpec((block_e, d), lambda i: (i, 0)),
            pl.BlockSpec((block_e, 1), lambda i: (i, 0)),
            pl.BlockSpec((block_e, 1), lambda i: (i, 0)),
            full((nrel, d)),
            full((3, d)),
            full((1, d)),
            full((d, d)),
            full((1, d)),
        ],
        out_specs=pl.BlockSpec((block_e, d), lambda i: (i, 0)),
        out_shape=jax.ShapeDtypeStruct((e, d), jnp.float32),
    )(hv, rel_b, ey, remb, ws, fb, w, gb)


def _tc_node_body(h, ps, pc, ent, fhw, fhb, gw, ghb, gruw, grub, out):
    d = h.shape[-1]
    cnt = pc[:, 0:1]
    aggr = ps[...] / jnp.maximum(cnt, 1.0)
    hh = h[...]
    e = ent[...]
    fw = fhw[...]
    u = (hh * (fw[0:1] + fw[1:2] * aggr + fw[2:3] * e)
         + aggr * (fw[3:4] + fw[5:6] * e) + fw[4:5] * e + fhb[...])
    upd = jnp.tanh(u)
    upd = upd + jnp.tanh(
        jnp.dot(upd, gw[...], preferred_element_type=jnp.float32) + ghb[...])
    zpre = (jnp.dot(hh, gruw[0:d, :], preferred_element_type=jnp.float32)
            + jnp.dot(upd, gruw[d:2 * d, :], preferred_element_type=jnp.float32)
            + grub[...])
    z = jax.nn.sigmoid(zpre)
    out[...] = (1.0 - z) * hh + z * upd


def _tc_node(h0, psums, pcnts, ent, fhw, fhb, gw, ghb, gruw, grub, block_n=2000):
    n, d = h0.shape
    assert n % block_n == 0
    grid = (n // block_n,)
    full = lambda shp: pl.BlockSpec(shp, lambda i: (0,) * len(shp))
    return pl.pallas_call(
        _tc_node_body,
        grid=grid,
        in_specs=[
            pl.BlockSpec((block_n, d), lambda i: (i, 0)),
            pl.BlockSpec((block_n, d), lambda i: (i, 0)),
            pl.BlockSpec((block_n, 16), lambda i: (i, 0)),
            pl.BlockSpec((block_n, d), lambda i: (i, 0)),
            full((6, d)),
            full((1, d)),
            full((d, d)),
            full((1, d)),
            full((2 * d, d)),
            full((1, d)),
        ],
        out_specs=pl.BlockSpec((block_n, d), lambda i: (i, 0)),
        out_shape=jax.ShapeDtypeStruct((n, d), jnp.float32),
    )(h0, psums, pcnts, ent, fhw, fhb, gw, ghb, gruw, grub)


def kernel(hidden, edges_y, selected_edges, ent_emb, rel_emb,
           f_msg_ws, f_msg_b, g_msg_W, g_msg_b,
           f_hid_ws, f_hid_b, g_hid_W, g_hid_b, gru_W, gru_b):
    _, n, d = hidden.shape
    e = selected_edges.shape[0]
    h0 = hidden[0]
    vi = selected_edges[:, 1]
    rl = selected_edges[:, 3]
    vj = selected_edges[:, 5]

    hv = _sc_gather(h0, vi)
    msg = _tc_edge(hv, rl.reshape(e, 1), edges_y.reshape(e, 1), rel_emb,
                   f_msg_ws, f_msg_b.reshape(1, d), g_msg_W,
                   g_msg_b.reshape(1, d))
    sums_p, cnts_p = _sc_segment(msg, vj)
    out = _tc_node(h0, sums_p[:n], cnts_p[:n], ent_emb, f_hid_ws,
                   f_hid_b.reshape(1, d), g_hid_W, g_hid_b.reshape(1, d),
                   gru_W, gru_b.reshape(1, d))
    return out.reshape(1, n, d)
